# baseline (speedup 1.0000x reference)
import functools

import jax
import jax.numpy as jnp
from jax import lax
from jax.experimental import pallas as pl
from jax.experimental.pallas import tpu as pltpu
from jax.experimental.pallas import tpu_sc as plsc

NUM_CORES = 2
NUM_SUBCORES = 16
NUM_WORKERS = NUM_CORES * NUM_SUBCORES
LANES = 16


def _wid():
    return lax.axis_index("s") * NUM_CORES + lax.axis_index("c")


@functools.lru_cache(maxsize=None)
def _format_table(V: int, D: int):
    n_full = V // 128
    rem = V % 128
    mesh = plsc.VectorSubcoreMesh(core_axis_name="c", subcore_axis_name="s")

    @functools.partial(
        pl.kernel,
        out_type=jax.ShapeDtypeStruct((V // 2, 128), jnp.float32),
        mesh=mesh,
        scratch_types=[
            pltpu.VMEM((D, 128), jnp.float32),
            pltpu.VMEM((D, 128), jnp.float32),
            pltpu.VMEM((64, 128), jnp.float32),
            pltpu.SemaphoreType.DMA,
            pltpu.SemaphoreType.DMA,
        ],
        compiler_params=pltpu.CompilerParams(needs_layout_passes=False),
    )
    def fmt(emb_t, tail_p, scratch, src0, src1, obuf, sem0, sem1):
        w = _wid()
        srcs = (src0, src1)
        sems = (sem0, sem1)
        iota = lax.iota(jnp.int32, LANES)
        n_iters = (n_full + NUM_WORKERS - 1) // NUM_WORKERS

        def transpose_block(src, n_pairs):
            def body(q4, carry):
                vals = []
                for dq in range(4):
                    q = q4 * 4 + dq
                    for m in range(8):
                        fvec = (m % 4) * 16 + iota
                        cvec = jnp.full((LANES,), 2 * q + (m // 4), jnp.int32)
                        vals.append(plsc.load_gather(src, [fvec, cvec]))
                for dq in range(4):
                    q = q4 * 4 + dq
                    for m in range(8):
                        obuf[q, pl.ds(m * 16, LANES)] = vals[dq * 8 + m]
                return carry

            lax.fori_loop(0, n_pairs // 4, body, 0)

        pltpu.async_copy(emb_t.at[:, pl.ds(w * 128, 128)], src0, sem0)

        def step(k2, carry):
            for b in range(2):
                k = k2 * 2 + b
                v = w + k * NUM_WORKERS

                @pl.when(v < n_full)
                def _():
                    vn = v + NUM_WORKERS

                    @pl.when(vn < n_full)
                    def _():
                        pltpu.async_copy(
                            emb_t.at[:, pl.ds(vn * 128, 128)],
                            srcs[1 - b],
                            sems[1 - b],
                        )

                    pltpu.make_async_copy(
                        emb_t.at[:, pl.ds(v * 128, 128)], srcs[b], sems[b]
                    ).wait()
                    transpose_block(srcs[b], 64)
                    pltpu.sync_copy(obuf, scratch.at[pl.ds(v * 64, 64)])

            return carry

        lax.fori_loop(0, (n_iters + 1) // 2, step, 0)

        if rem:
            @pl.when(w == 0)
            def _():
                pltpu.sync_copy(tail_p, src0)
                transpose_block(src0, rem // 2)
                pltpu.sync_copy(
                    obuf.at[pl.ds(0, rem // 2)],
                    scratch.at[pl.ds(n_full * 64, rem // 2)],
                )

    return fmt


@functools.lru_cache(maxsize=None)
def _gather_embed(S: int, T: int, D: int, V: int):
    s_per_w = S // NUM_WORKERS
    n_sb = s_per_w // 128
    mesh = plsc.VectorSubcoreMesh(core_axis_name="c", subcore_axis_name="s")

    DEPTH = 3

    @functools.partial(
        pl.kernel,
        out_type=jax.ShapeDtypeStruct((T, D, S), jnp.float32),
        mesh=mesh,
        scratch_types=[
            pltpu.VMEM((128, T), jnp.int32),
            pltpu.VMEM((DEPTH, 128), jnp.int32),
            pltpu.VMEM((128, 128), jnp.float32),
            pltpu.VMEM((128, 128), jnp.float32),
            pltpu.VMEM((128, 128), jnp.float32),
            pltpu.VMEM((D, 128), jnp.float32),
            pltpu.VMEM((D, 128), jnp.float32),
            pltpu.VMEM((D, 128), jnp.float32),
            pltpu.SemaphoreType.DMA,
            pltpu.SemaphoreType.DMA,
            pltpu.SemaphoreType.DMA,
            pltpu.SemaphoreType.DMA,
            pltpu.SemaphoreType.DMA,
            pltpu.SemaphoreType.DMA,
        ],
        compiler_params=pltpu.CompilerParams(needs_layout_passes=False),
    )
    def gat(
        x_hbm, tab, out, xbuf, pball, rb0, rb1, rb2, tb0, tb1, tb2,
        g0, g1, g2, w0, w1, w2,
    ):
        w = _wid()
        rbs = (rb0, rb1, rb2)
        tbs = (tb0, tb1, tb2)
        gsems = (g0, g1, g2)
        wsems = (w0, w1, w2)
        iota = lax.iota(jnp.int32, LANES)
        kvecs = [m * 16 + iota for m in range(8)]

        def extract(t, q):
            tvec = jnp.full((LANES,), t, jnp.int32)
            ivs = [plsc.load_gather(xbuf, [kvecs[m], tvec]) for m in range(8)]
            hw = [lax.shift_left(lax.bitwise_and(iv, 1), 6) for iv in ivs]
            for m in range(8):
                pball[q, pl.ds(m * 16, LANES)] = lax.shift_right_logical(
                    ivs[m], 1
                )
            return tuple(hw)

        def start_gather(q):
            pltpu.async_copy(tab.at[pball.at[q]], rbs[q], gsems[q])

        def wait_gather(q):
            pltpu.make_async_copy(tab.at[pball.at[q]], rbs[q], gsems[q]).wait()

        def transpose(q, hw):
            rb = rbs[q]
            tb = tbs[q]
            for j4 in range(D // 4):
                vals = []
                for dj in range(4):
                    j = j4 * 4 + dj
                    for m in range(8):
                        vals.append(plsc.load_gather(rb, [kvecs[m], hw[m] + j]))
                for dj in range(4):
                    j = j4 * 4 + dj
                    for m in range(8):
                        tb[j, pl.ds(m * 16, LANES)] = vals[dj * 8 + m]

        def start_write(t, q, s_base):
            pltpu.async_copy(tbs[q], out.at[t, :, pl.ds(s_base, 128)], wsems[q])

        def drain_write(t, q, s_base):
            pltpu.make_async_copy(
                tbs[q], out.at[t, :, pl.ds(s_base, 128)], wsems[q]
            ).wait()

        def sb_step(sb, carry):
            s_base = w * s_per_w + sb * 128
            pltpu.sync_copy(x_hbm.at[pl.ds(s_base, 128)], xbuf)
            hw = []
            for q in range(DEPTH):
                hw.append(extract(jnp.minimum(q, T - 1), q))
                if q < T:
                    start_gather(q)
            hw = tuple(hw)

            def step(t3, hw_c):
                hws = list(hw_c)
                for q in range(DEPTH):
                    t = t3 * DEPTH + q

                    @pl.when(t < T)
                    def _():
                        wait_gather(q)

                    @pl.when(jnp.logical_and(t >= DEPTH, t < T))
                    def _():
                        drain_write(t, q, s_base)

                    @pl.when(t < T)
                    def _():
                        transpose(q, hws[q])
                        start_write(t, q, s_base)

                    hw_nxt = extract(jnp.minimum(t + DEPTH, T - 1), q)

                    @pl.when(t + DEPTH < T)
                    def _():
                        start_gather(q)

                    hws[q] = hw_nxt
                return tuple(hws)

            n_t3 = (T + DEPTH - 1) // DEPTH
            lax.fori_loop(0, n_t3, step, hw)
            for q in range(min(DEPTH, T)):
                drain_write(0, q, s_base)
            return carry

        lax.fori_loop(0, n_sb, sb_step, 0)

    return gat


def kernel(x, emb):
    S, T = x.shape
    V, D = emb.shape
    scratch = jnp.reshape(emb, (V // 2, 2 * D))
    out3 = _gather_embed(S, T, D, V)(x.astype(jnp.int32), scratch)
    return jnp.transpose(out3, (2, 0, 1))


# speedup vs baseline: 1.0105x; 1.0105x over previous
import functools

import jax
import jax.numpy as jnp
from jax import lax
from jax.experimental import pallas as pl
from jax.experimental.pallas import tpu as pltpu
from jax.experimental.pallas import tpu_sc as plsc

NUM_CORES = 2
NUM_SUBCORES = 16
NUM_WORKERS = NUM_CORES * NUM_SUBCORES
LANES = 16


def _wid():
    return lax.axis_index("s") * NUM_CORES + lax.axis_index("c")


@functools.lru_cache(maxsize=None)
def _gather_embed(S: int, T: int, D: int, V: int):
    s_per_w = S // NUM_WORKERS
    n_sb = s_per_w // 128
    DEPTH = 2
    TT = 2
    n_c = T // TT
    mesh = plsc.VectorSubcoreMesh(core_axis_name="c", subcore_axis_name="s")

    @functools.partial(
        pl.kernel,
        out_type=jax.ShapeDtypeStruct((T, D // 8, S // 128, 8, 128), jnp.float32),
        mesh=mesh,
        scratch_types=[
            pltpu.VMEM((128, T), jnp.int32),
            pltpu.VMEM((DEPTH, TT * 128), jnp.int32),
            pltpu.VMEM((TT * 128, D), jnp.float32),
            pltpu.VMEM((TT * 128, D), jnp.float32),
            pltpu.VMEM((TT, D // 8, 8, 128), jnp.float32),
            pltpu.VMEM((TT, D // 8, 8, 128), jnp.float32),
            pltpu.SemaphoreType.DMA,
            pltpu.SemaphoreType.DMA,
            pltpu.SemaphoreType.DMA,
            pltpu.SemaphoreType.DMA,
        ],
        compiler_params=pltpu.CompilerParams(
            needs_layout_passes=False, use_tc_tiling_on_sc=False
        ),
    )
    def gat(x_hbm, tab, out, xbuf, pball, rb0, rb1, tb0, tb1, g0, g1, w0, w1):
        w = _wid()
        rbs = (rb0, rb1)
        tbs = (tb0, tb1)
        gsems = (g0, g1)
        wsems = (w0, w1)
        iota = lax.iota(jnp.int32, LANES)
        kvecs = [m * 16 + iota for m in range(8)]
        jvecs = [jnp.full((LANES,), j, jnp.int32) for j in range(D)]

        def extract(c, q):
            for tt in range(TT):
                t = c * TT + tt
                tvec = jnp.full((LANES,), t, jnp.int32)
                ivs = [
                    plsc.load_gather(xbuf, [kvecs[m], tvec]) for m in range(8)
                ]
                for m in range(8):
                    pball[q, pl.ds(tt * 128 + m * 16, LANES)] = ivs[m]

        def start_gather(q):
            pltpu.async_copy(tab.at[pball.at[q]], rbs[q], gsems[q])

        def wait_gather(q):
            pltpu.make_async_copy(tab.at[pball.at[q]], rbs[q], gsems[q]).wait()

        def transpose(q):
            rb = rbs[q]
            tb = tbs[q]
            for tt in range(TT):
                for j4 in range(D // 4):
                    vals = []
                    for dj in range(4):
                        j = j4 * 4 + dj
                        for m in range(8):
                            vals.append(
                                plsc.load_gather(
                                    rb, [tt * 128 + kvecs[m], jvecs[j]]
                                )
                            )
                    for dj in range(4):
                        j = j4 * 4 + dj
                        for m in range(8):
                            tb[tt, j // 8, j % 8, pl.ds(m * 16, LANES)] = vals[
                                dj * 8 + m
                            ]

        def start_write(c, q, sbi):
            for tt in range(TT):
                pltpu.async_copy(
                    tbs[q].at[tt], out.at[c * TT + tt, :, sbi], wsems[q]
                )

        def drain_write(q, sbi):
            for tt in range(TT):
                pltpu.make_async_copy(
                    tbs[q].at[tt], out.at[tt, :, sbi], wsems[q]
                ).wait()

        def sb_step(sb, carry):
            sbi = (w * s_per_w) // 128 + sb
            pltpu.sync_copy(x_hbm.at[pl.ds(sbi * 128, 128)], xbuf)
            for q in range(DEPTH):
                extract(jnp.int32(q), q)
                start_gather(q)

            def step(c2, carry2):
                for q in range(DEPTH):
                    c = c2 * DEPTH + q

                    @pl.when(c < n_c)
                    def _():
                        wait_gather(q)

                    @pl.when(jnp.logical_and(c >= DEPTH, c < n_c))
                    def _():
                        drain_write(q, sbi)

                    @pl.when(c < n_c)
                    def _():
                        transpose(q)
                        start_write(c, q, sbi)

                    @pl.when(c + DEPTH < n_c)
                    def _():
                        extract(c + DEPTH, q)
                        start_gather(q)

                return carry2

            lax.fori_loop(0, (n_c + DEPTH - 1) // DEPTH, step, 0)
            for q in range(min(DEPTH, n_c)):
                drain_write(q, sbi)
            return carry

        lax.fori_loop(0, n_sb, sb_step, 0)

    return gat


def kernel(x, emb):
    S, T = x.shape
    V, D = emb.shape
    out5 = _gather_embed(S, T, D, V)(x.astype(jnp.int32), emb)
    return jnp.transpose(out5, (2, 4, 0, 1, 3)).reshape(S, T, D)
